# baseline (speedup 1.0000x reference)
import functools

import numpy as np
import jax
import jax.numpy as jnp
from jax import lax
from jax.experimental import pallas as pl
from jax.experimental.pallas import tpu as pltpu
from jax.experimental.pallas import tpu_sc as plsc

D = 256
HID = 256
OUT = 64

NW = 32
CHUNK = 80
CHUNKS_PER_W = 10
ROWS_PER_W = CHUNK * CHUNKS_PER_W
N_PAD = NW * ROWS_PER_W

TC_BLOCK = 1024


def _sc_gather(imr_hbm, gr_hbm, idx_hbm, x1_hbm, x2_hbm,
               idx_v, b1a, b1b, b2a, b2b, o1a, o1b, o2a, o2b,
               sem_g, sem_wa, sem_wb):
    wid = lax.axis_index("s") * 2 + lax.axis_index("c")
    base = wid * ROWS_PER_W
    bufs = ((b1a, b1b), (b2a, b2b))
    obufs = ((o1a, o1b), (o2a, o2b))
    outs = (x1_hbm, x2_hbm)
    sem_w = (sem_wa, sem_wb)
    pltpu.sync_copy(idx_hbm.at[pl.ds(base, ROWS_PER_W)], idx_v)

    def pack_chunk(fbuf, obuf):
        def pair_body(k, _):
            for t in range(0, D, 16):
                a = fbuf[2 * k, pl.ds(t, 16)]
                b = fbuf[2 * k + 1, pl.ds(t, 16)]
                p = plsc.pack(a, b, format=plsc.PackFormat.INTERLEAVED)
                obuf[k, pl.ds(t, 16)] = plsc.bitcast(p, jnp.int32)
            return _
        lax.fori_loop(0, CHUNK // 2, pair_body, 0)

    idx0 = idx_v.at[pl.ds(0, CHUNK)]
    gcur = [pltpu.async_copy(imr_hbm.at[idx0], bufs[0][0], sem_g),
            pltpu.async_copy(gr_hbm.at[idx0], bufs[1][0], sem_g)]
    pending = []
    for c in range(CHUNKS_PER_W):
        cur = c % 2
        nxt = (c + 1) % 2
        for d in gcur:
            d.wait()
        if c + 1 < CHUNKS_PER_W:
            idx_c = idx_v.at[pl.ds((c + 1) * CHUNK, CHUNK)]
            gcur = [pltpu.async_copy(imr_hbm.at[idx_c], bufs[0][nxt], sem_g),
                    pltpu.async_copy(gr_hbm.at[idx_c], bufs[1][nxt], sem_g)]
        if len(pending) >= 2:
            for d in pending.pop(0):
                d.wait()
        pack_chunk(bufs[0][cur], obufs[0][cur])
        pack_chunk(bufs[1][cur], obufs[1][cur])
        row0 = wid * (ROWS_PER_W // 2) + c * (CHUNK // 2)
        pending.append([
            pltpu.async_copy(obufs[0][cur], x1_hbm.at[pl.ds(row0, CHUNK // 2)],
                             sem_w[cur]),
            pltpu.async_copy(obufs[1][cur], x2_hbm.at[pl.ds(row0, CHUNK // 2)],
                             sem_w[cur]),
        ])
    for grp in pending:
        for d in grp:
            d.wait()


def _gather_rows(imr, gr, idx_pad):
    mesh = plsc.VectorSubcoreMesh(core_axis_name="c", subcore_axis_name="s")
    f = pl.kernel(
        _sc_gather,
        out_type=[
            jax.ShapeDtypeStruct((N_PAD // 2, D), jnp.int32),
            jax.ShapeDtypeStruct((N_PAD // 2, D), jnp.int32),
        ],
        mesh=mesh,
        scratch_types=[
            pltpu.VMEM((ROWS_PER_W,), jnp.int32),
            pltpu.VMEM((CHUNK, D), jnp.float32),
            pltpu.VMEM((CHUNK, D), jnp.float32),
            pltpu.VMEM((CHUNK, D), jnp.float32),
            pltpu.VMEM((CHUNK, D), jnp.float32),
            pltpu.VMEM((CHUNK // 2, D), jnp.int32),
            pltpu.VMEM((CHUNK // 2, D), jnp.int32),
            pltpu.VMEM((CHUNK // 2, D), jnp.int32),
            pltpu.VMEM((CHUNK // 2, D), jnp.int32),
            pltpu.SemaphoreType.DMA,
            pltpu.SemaphoreType.DMA,
            pltpu.SemaphoreType.DMA,
        ],
        compiler_params=pltpu.CompilerParams(needs_layout_passes=False),
    )
    return f(imr, gr, idx_pad)


def _mlp_body(x1_ref, x2_ref, w1a_ref, w1b_ref, w2_ref, b1_ref, b2_ref, o_ref):
    def as_bf16(ref):
        return pltpu.bitcast(ref[...], jnp.bfloat16)
    h = jnp.dot(as_bf16(x1_ref), w1a_ref[...],
                preferred_element_type=jnp.float32)
    h += jnp.dot(as_bf16(x2_ref), w1b_ref[...],
                 preferred_element_type=jnp.float32)
    h = jnp.maximum(h + b1_ref[...], 0.0)
    o_ref[...] = (
        jnp.dot(h, w2_ref[...], preferred_element_type=jnp.float32)
        + b2_ref[...]
    )


def _mlp(x1, x2, w1a, w1b, w2, b1r, b2r, n):
    grid = (pl.cdiv(n, TC_BLOCK),)
    return pl.pallas_call(
        _mlp_body,
        grid=grid,
        in_specs=[
            pl.BlockSpec((TC_BLOCK // 2, D), lambda i: (i, 0)),
            pl.BlockSpec((TC_BLOCK // 2, D), lambda i: (i, 0)),
            pl.BlockSpec((D, HID), lambda i: (0, 0)),
            pl.BlockSpec((D, HID), lambda i: (0, 0)),
            pl.BlockSpec((HID, OUT), lambda i: (0, 0)),
            pl.BlockSpec((1, HID), lambda i: (0, 0)),
            pl.BlockSpec((1, OUT), lambda i: (0, 0)),
        ],
        out_specs=pl.BlockSpec((TC_BLOCK, OUT), lambda i: (i, 0)),
        out_shape=jax.ShapeDtypeStruct((n, OUT), jnp.float32),
    )(x1, x2, w1a, w1b, w2, b1r, b2r)


def kernel(input_molecule_representations, graph_representations,
           graphs_requiring_node_choices, W1, b1, W2, b2):
    n_sel = graphs_requiring_node_choices.shape[0]
    idx = graphs_requiring_node_choices.astype(jnp.int32)
    idx_pad = jnp.concatenate(
        [idx, jnp.zeros((N_PAD - n_sel,), jnp.int32)])
    x1, x2 = _gather_rows(
        input_molecule_representations, graph_representations, idx_pad)
    w1a = W1[:D].astype(jnp.bfloat16)
    w1b = W1[D:].astype(jnp.bfloat16)
    return _mlp(x1, x2, w1a, w1b, W2,
                b1.reshape(1, HID), b2.reshape(1, OUT), n_sel)


# speedup vs baseline: 1.3407x; 1.3407x over previous
import jax
import jax.numpy as jnp
from jax import lax
from jax.experimental import pallas as pl
from jax.experimental.pallas import tpu as pltpu
from jax.experimental.pallas import tpu_sc as plsc

D = 256
HID = 256
OUT = 64

NW = 32
CHUNK = 56
CHUNKS_PER_W = 7
ROWS_PER_W = CHUNK * CHUNKS_PER_W
N_HALF = NW * ROWS_PER_W
N_SPLIT = (12544, 12456)
TC_BLOCKS = (1568, 1384)


def _sc_gather(imr_hbm, gr_hbm, idx_hbm, x_hbm,
               idx_v, b1a, b1b, b2a, b2b, sem_g, sem_wa, sem_wb):
    wid = lax.axis_index("s") * 2 + lax.axis_index("c")
    base = wid * ROWS_PER_W
    bufs1 = (b1a, b1b)
    bufs2 = (b2a, b2b)
    sem_w = (sem_wa, sem_wb)
    pltpu.sync_copy(idx_hbm.at[pl.ds(base, ROWS_PER_W)], idx_v)
    idx0 = idx_v.at[pl.ds(0, CHUNK)]
    gcur = [pltpu.async_copy(imr_hbm.at[idx0], bufs1[0], sem_g),
            pltpu.async_copy(gr_hbm.at[idx0], bufs2[0], sem_g)]
    pending = []
    for c in range(CHUNKS_PER_W):
        cur = c % 2
        nxt = (c + 1) % 2
        for d in gcur:
            d.wait()
        row0 = base + c * CHUNK
        pending.append([
            pltpu.async_copy(
                bufs1[cur], x_hbm.at[pl.ds(row0, CHUNK), pl.ds(0, D)],
                sem_w[cur]),
            pltpu.async_copy(
                bufs2[cur], x_hbm.at[pl.ds(row0, CHUNK), pl.ds(D, D)],
                sem_w[cur]),
        ])
        if c + 1 < CHUNKS_PER_W:
            if len(pending) >= 2:
                for d in pending.pop(0):
                    d.wait()
            idx_c = idx_v.at[pl.ds((c + 1) * CHUNK, CHUNK)]
            gcur = [pltpu.async_copy(imr_hbm.at[idx_c], bufs1[nxt], sem_g),
                    pltpu.async_copy(gr_hbm.at[idx_c], bufs2[nxt], sem_g)]
    for grp in pending:
        for d in grp:
            d.wait()


def _gather_rows(imr, gr, idx_half):
    mesh = plsc.VectorSubcoreMesh(core_axis_name="c", subcore_axis_name="s")
    f = pl.kernel(
        _sc_gather,
        out_type=jax.ShapeDtypeStruct((N_HALF, 2 * D), jnp.float32),
        mesh=mesh,
        scratch_types=[
            pltpu.VMEM((ROWS_PER_W,), jnp.int32),
            pltpu.VMEM((CHUNK, D), jnp.float32),
            pltpu.VMEM((CHUNK, D), jnp.float32),
            pltpu.VMEM((CHUNK, D), jnp.float32),
            pltpu.VMEM((CHUNK, D), jnp.float32),
            pltpu.SemaphoreType.DMA,
            pltpu.SemaphoreType.DMA,
            pltpu.SemaphoreType.DMA,
        ],
    )
    return f(imr, gr, idx_half)


def _mlp_body(x_ref, w1_ref, w2_ref, b1_ref, b2_ref, o_ref):
    h = jnp.dot(x_ref[...], w1_ref[...], preferred_element_type=jnp.float32)
    h = jnp.maximum(h + b1_ref[...], 0.0)
    o_ref[...] = (
        jnp.dot(h, w2_ref[...], preferred_element_type=jnp.float32)
        + b2_ref[...]
    )


def _mlp(x, w1, w2, b1r, b2r, n, blk):
    grid = (n // blk,)
    return pl.pallas_call(
        _mlp_body,
        grid=grid,
        in_specs=[
            pl.BlockSpec((blk, 2 * D), lambda i: (i, 0)),
            pl.BlockSpec((2 * D, HID), lambda i: (0, 0)),
            pl.BlockSpec((HID, OUT), lambda i: (0, 0)),
            pl.BlockSpec((1, HID), lambda i: (0, 0)),
            pl.BlockSpec((1, OUT), lambda i: (0, 0)),
        ],
        out_specs=pl.BlockSpec((blk, OUT), lambda i: (i, 0)),
        out_shape=jax.ShapeDtypeStruct((n, OUT), jnp.float32),
    )(x, w1, w2, b1r, b2r)


def kernel(input_molecule_representations, graph_representations,
           graphs_requiring_node_choices, W1, b1, W2, b2):
    n_sel = graphs_requiring_node_choices.shape[0]
    idx = graphs_requiring_node_choices.astype(jnp.int32)
    idx_pad = jnp.concatenate(
        [idx, jnp.zeros((2 * N_HALF - n_sel,), jnp.int32)])
    b1r = b1.reshape(1, HID)
    b2r = b2.reshape(1, OUT)
    outs = []
    for h in range(2):
        x = _gather_rows(
            input_molecule_representations, graph_representations,
            idx_pad[h * N_HALF:(h + 1) * N_HALF])
        outs.append(
            _mlp(x, W1, W2, b1r, b2r, N_SPLIT[h], TC_BLOCKS[h]))
    return jnp.concatenate(outs, axis=0)


# speedup vs baseline: 1.3871x; 1.0346x over previous
import jax
import jax.numpy as jnp
from jax import lax
from jax.experimental import pallas as pl
from jax.experimental.pallas import tpu as pltpu
from jax.experimental.pallas import tpu_sc as plsc

D = 256
HID = 256
OUT = 64

NW = 32
CHUNK = 56
CHUNKS_PER_W = 7
ROWS_PER_W = CHUNK * CHUNKS_PER_W
N_HALF = NW * ROWS_PER_W
N_OUT = 25000
TC_BLOCK = 1568


def _sc_gather(imr_hbm, gr_hbm, idx_hbm, x_hbm,
               idx_v, b1a, b1b, b2a, b2b, sem_g, sem_wa, sem_wb):
    wid = lax.axis_index("s") * 2 + lax.axis_index("c")
    base = wid * ROWS_PER_W
    bufs1 = (b1a, b1b)
    bufs2 = (b2a, b2b)
    sem_w = (sem_wa, sem_wb)
    pltpu.sync_copy(idx_hbm.at[pl.ds(base, ROWS_PER_W)], idx_v)
    idx0 = idx_v.at[pl.ds(0, CHUNK)]
    gcur = [pltpu.async_copy(imr_hbm.at[idx0], bufs1[0], sem_g),
            pltpu.async_copy(gr_hbm.at[idx0], bufs2[0], sem_g)]
    pending = []
    for c in range(CHUNKS_PER_W):
        cur = c % 2
        nxt = (c + 1) % 2
        for d in gcur:
            d.wait()
        row0 = base + c * CHUNK
        pending.append([
            pltpu.async_copy(
                bufs1[cur], x_hbm.at[pl.ds(row0, CHUNK), pl.ds(0, D)],
                sem_w[cur]),
            pltpu.async_copy(
                bufs2[cur], x_hbm.at[pl.ds(row0, CHUNK), pl.ds(D, D)],
                sem_w[cur]),
        ])
        if c + 1 < CHUNKS_PER_W:
            if len(pending) >= 2:
                for d in pending.pop(0):
                    d.wait()
            idx_c = idx_v.at[pl.ds((c + 1) * CHUNK, CHUNK)]
            gcur = [pltpu.async_copy(imr_hbm.at[idx_c], bufs1[nxt], sem_g),
                    pltpu.async_copy(gr_hbm.at[idx_c], bufs2[nxt], sem_g)]
    for grp in pending:
        for d in grp:
            d.wait()


def _gather_rows(imr, gr, idx_half):
    mesh = plsc.VectorSubcoreMesh(core_axis_name="c", subcore_axis_name="s")
    f = pl.kernel(
        _sc_gather,
        out_type=jax.ShapeDtypeStruct((N_HALF, 2 * D), jnp.float32),
        mesh=mesh,
        scratch_types=[
            pltpu.VMEM((ROWS_PER_W,), jnp.int32),
            pltpu.VMEM((CHUNK, D), jnp.float32),
            pltpu.VMEM((CHUNK, D), jnp.float32),
            pltpu.VMEM((CHUNK, D), jnp.float32),
            pltpu.VMEM((CHUNK, D), jnp.float32),
            pltpu.SemaphoreType.DMA,
            pltpu.SemaphoreType.DMA,
            pltpu.SemaphoreType.DMA,
        ],
    )
    return f(imr, gr, idx_half)


def _mlp_body(x_ref, w1_ref, w2_ref, b1_ref, b2_ref, oprev_ref, o_ref):
    del oprev_ref
    h = jnp.dot(x_ref[...], w1_ref[...], preferred_element_type=jnp.float32)
    h = jnp.maximum(h + b1_ref[...], 0.0)
    o_ref[...] = (
        jnp.dot(h, w2_ref[...], preferred_element_type=jnp.float32)
        + b2_ref[...]
    )


def _mlp(x, w1, w2, b1r, b2r, o_prev, half):
    off = half * (N_HALF // TC_BLOCK)
    return pl.pallas_call(
        _mlp_body,
        grid=(N_HALF // TC_BLOCK,),
        in_specs=[
            pl.BlockSpec((TC_BLOCK, 2 * D), lambda i: (i, 0)),
            pl.BlockSpec((2 * D, HID), lambda i: (0, 0)),
            pl.BlockSpec((HID, OUT), lambda i: (0, 0)),
            pl.BlockSpec((1, HID), lambda i: (0, 0)),
            pl.BlockSpec((1, OUT), lambda i: (0, 0)),
            pl.BlockSpec((TC_BLOCK, OUT), lambda i: (i + off, 0)),
        ],
        out_specs=pl.BlockSpec((TC_BLOCK, OUT), lambda i: (i + off, 0)),
        out_shape=jax.ShapeDtypeStruct((N_OUT, OUT), jnp.float32),
        input_output_aliases={5: 0},
    )(x, w1, w2, b1r, b2r, o_prev)


def kernel(input_molecule_representations, graph_representations,
           graphs_requiring_node_choices, W1, b1, W2, b2):
    n_sel = graphs_requiring_node_choices.shape[0]
    idx = graphs_requiring_node_choices.astype(jnp.int32)
    idx_pad = jnp.concatenate(
        [idx, jnp.zeros((2 * N_HALF - n_sel,), jnp.int32)])
    b1r = b1.reshape(1, HID)
    b2r = b2.reshape(1, OUT)
    out = jnp.zeros((N_OUT, OUT), jnp.float32)
    for h in range(2):
        x = _gather_rows(
            input_molecule_representations, graph_representations,
            idx_pad[h * N_HALF:(h + 1) * N_HALF])
        out = _mlp(x, W1, W2, b1r, b2r, out, h)
    return out
